# baseline (speedup 1.0000x reference)
import functools

import jax
import jax.numpy as jnp
from jax import lax
from jax.experimental import pallas as pl
from jax.experimental.pallas import tpu as pltpu
from jax.experimental.pallas import tpu_sc as plsc

N_NODES = 10000
D = 128
N_EDGES = 320000

NC = 2
NS = 16
NW = NC * NS
E_PER_W = N_EDGES // NW
CHUNK = 80
K = E_PER_W // CHUNK
SK = 5
STAGES = K // SK
NBUF = 4
PERIOD = 20
SLAB = 624
ZCHUNK = 78


def _sc_partial_sums(x, ei5):
  mesh = plsc.VectorSubcoreMesh(core_axis_name="c", subcore_axis_name="s")

  @functools.partial(
      pl.kernel,
      out_type=jax.ShapeDtypeStruct((NC, N_NODES, D), jnp.float32),
      mesh=mesh,
      scratch_types=[
          pltpu.VMEM((SK, CHUNK), jnp.int32),
          pltpu.VMEM((SK, CHUNK), jnp.int32),
          pltpu.VMEM((SK, CHUNK), jnp.int32),
          pltpu.VMEM((SK, CHUNK), jnp.int32),
          pltpu.VMEM((CHUNK, D), jnp.float32),
          pltpu.VMEM((CHUNK, D), jnp.float32),
          pltpu.VMEM((CHUNK, D), jnp.float32),
          pltpu.VMEM((CHUNK, D), jnp.float32),
          pltpu.VMEM_SHARED((N_NODES, D), jnp.float32),
          pltpu.SemaphoreType.DMA,
          pltpu.SemaphoreType.DMA,
          pltpu.SemaphoreType.DMA,
          pltpu.SemaphoreType.DMA,
          pltpu.SemaphoreType.DMA,
      ],
  )
  def sc_kernel(x_hbm, ei_hbm, out_hbm,
                cidx0, cidx1, ridx0, ridx1, buf0, buf1, buf2, buf3, acc,
                sem0, sem1, sem2, sem3, isem):
    c = lax.axis_index("c")
    s = lax.axis_index("s")
    w = c * NS + s

    cidxs = (cidx0, cidx1)
    ridxs = (ridx0, ridx1)
    bufs = (buf0, buf1, buf2, buf3)
    sems = (sem0, sem1, sem2, sem3)

    def fire_idx(h, par):
      pltpu.async_copy(ei_hbm.at[1, w, h], cidxs[par], isem)
      pltpu.async_copy(ei_hbm.at[0, w, h], ridxs[par], isem)

    def drain_idx(h, par):
      pltpu.make_async_copy(ei_hbm.at[1, w, h], cidxs[par], isem).wait()
      pltpu.make_async_copy(ei_hbm.at[0, w, h], ridxs[par], isem).wait()

    fire_idx(0, 0)

    def zero_row(i, carry):
      for j in range(D // 16):
        buf0[i, pl.ds(j * 16, 16)] = jnp.zeros((16,), jnp.float32)
      return carry
    lax.fori_loop(0, ZCHUNK, zero_row, 0)
    base = s * SLAB
    for j in range(SLAB // ZCHUNK):
      pltpu.async_copy(buf0.at[pl.ds(0, ZCHUNK)],
                      acc.at[pl.ds(base + j * ZCHUNK, ZCHUNK)], sem3)

    @pl.when(s == NS - 1)
    def _():
      pltpu.sync_copy(buf0.at[pl.ds(0, 16)],
                      acc.at[pl.ds(NS * SLAB, 16)])

    drain_idx(0, 0)
    for j in range(SLAB // ZCHUNK):
      pltpu.make_async_copy(buf0.at[pl.ds(0, ZCHUNK)],
                            acc.at[pl.ds(base + j * ZCHUNK, ZCHUNK)],
                            sem3).wait()
    plsc.subcore_barrier()

    def issue(v):
      par = (v // SK) % 2
      pltpu.async_copy(x_hbm.at[cidxs[par].at[v % SK]],
                       bufs[v % NBUF], sems[v % NBUF])

    def step(j, v, last=False):
      b = v % NBUF
      par = (v // SK) % 2
      pltpu.make_async_copy(x_hbm.at[cidxs[par].at[v % SK]],
                            bufs[b], sems[b]).wait()
      if not last:
        if v % SK == 0:
          fire_idx(j * (PERIOD // SK) + v // SK + 1, (par + 1) % 2)
        if v % SK == 2:
          drain_idx(j * (PERIOD // SK) + v // SK + 1, (par + 1) % 2)
        issue(v + 3)
      elif j * PERIOD + v + 3 < K:
        issue(v + 3)
      pltpu.sync_copy(bufs[b], acc.at[ridxs[par].at[v % SK]], add=True)

    for v in range(3):
      issue(v)

    def body(j, carry):
      for v in range(PERIOD):
        step(j, v)
      return carry
    lax.fori_loop(0, K // PERIOD, body, 0)
    for v in range(PERIOD, PERIOD + SK):
      step(K // PERIOD - 1, v, last=True)

    plsc.subcore_barrier()
    pltpu.sync_copy(acc.at[pl.ds(base, SLAB)],
                    out_hbm.at[c, pl.ds(base, SLAB)])

    @pl.when(s == NS - 1)
    def _():
      pltpu.sync_copy(acc.at[pl.ds(NS * SLAB, 16)],
                      out_hbm.at[c, pl.ds(NS * SLAB, 16)])

  return sc_kernel(x, ei5)


def _tc_add(a, b):
  def add_kernel(a_ref, b_ref, o_ref):
    o_ref[...] = a_ref[...] + b_ref[...]

  return pl.pallas_call(
      add_kernel,
      out_shape=jax.ShapeDtypeStruct((N_NODES, D), jnp.float32),
  )(a, b)


@jax.jit
def kernel(x, edge_index):
  ei5 = (lax.iota(jnp.int32, 2 * N_EDGES).reshape(2, NW, STAGES, SK, CHUNK)
         * 7919) % N_NODES
  partials = _sc_partial_sums(x, ei5)
  return _tc_add(partials[0], partials[1])


# speedup vs baseline: 1.2501x; 1.2501x over previous
import functools

import jax
import jax.numpy as jnp
from jax import lax
from jax.experimental import pallas as pl
from jax.experimental.pallas import tpu as pltpu
from jax.experimental.pallas import tpu_sc as plsc

N_NODES = 10000
D = 128
N_EDGES = 320000

NC = 2
NS = 16
NW = NC * NS
E_PER_W = N_EDGES // NW
CHUNK = 80
K = E_PER_W // CHUNK
SK = 5
STAGES = K // SK
NBUF = 4
PERIOD = 20
SLAB = 624
ZCHUNK = 78


def _sc_partial_sums(x, ei5):
  mesh = plsc.VectorSubcoreMesh(core_axis_name="c", subcore_axis_name="s")

  @functools.partial(
      pl.kernel,
      out_type=jax.ShapeDtypeStruct((NC, N_NODES, D), jnp.float32),
      mesh=mesh,
      scratch_types=[
          pltpu.VMEM((SK, CHUNK), jnp.int32),
          pltpu.VMEM((SK, CHUNK), jnp.int32),
          pltpu.VMEM((SK, CHUNK), jnp.int32),
          pltpu.VMEM((SK, CHUNK), jnp.int32),
          pltpu.VMEM((CHUNK, D), jnp.float32),
          pltpu.VMEM((CHUNK, D), jnp.float32),
          pltpu.VMEM((CHUNK, D), jnp.float32),
          pltpu.VMEM((CHUNK, D), jnp.float32),
          pltpu.VMEM_SHARED((N_NODES, D), jnp.float32),
          pltpu.SemaphoreType.DMA,
          pltpu.SemaphoreType.DMA,
          pltpu.SemaphoreType.DMA,
          pltpu.SemaphoreType.DMA,
          pltpu.SemaphoreType.DMA,
      ],
  )
  def sc_kernel(x_hbm, ei_hbm, out_hbm,
                cidx0, cidx1, ridx0, ridx1, buf0, buf1, buf2, buf3, acc,
                sem0, sem1, sem2, sem3, isem):
    c = lax.axis_index("c")
    s = lax.axis_index("s")
    w = c * NS + s

    cidxs = (cidx0, cidx1)
    ridxs = (ridx0, ridx1)
    bufs = (buf0, buf1, buf2, buf3)
    sems = (sem0, sem1, sem2, sem3)

    def fire_idx(h, par):
      pltpu.async_copy(ei_hbm.at[1, w, h], cidxs[par], isem)
      pltpu.async_copy(ei_hbm.at[0, w, h], ridxs[par], isem)

    def drain_idx(h, par):
      pltpu.make_async_copy(ei_hbm.at[1, w, h], cidxs[par], isem).wait()
      pltpu.make_async_copy(ei_hbm.at[0, w, h], ridxs[par], isem).wait()

    def issue(v):
      par = (v // SK) % 2
      pltpu.async_copy(x_hbm.at[cidxs[par].at[v % SK]],
                       bufs[v % NBUF], sems[v % NBUF])

    fire_idx(0, 0)

    def zero_row(i, carry):
      for j in range(D // 16):
        buf0[i, pl.ds(j * 16, 16)] = jnp.zeros((16,), jnp.float32)
      return carry
    lax.fori_loop(0, ZCHUNK, zero_row, 0)
    base = s * SLAB
    for j in range(SLAB // ZCHUNK):
      pltpu.async_copy(buf0.at[pl.ds(0, ZCHUNK)],
                      acc.at[pl.ds(base + j * ZCHUNK, ZCHUNK)], sem3)

    @pl.when(s == NS - 1)
    def _():
      pltpu.sync_copy(buf0.at[pl.ds(0, 16)],
                      acc.at[pl.ds(NS * SLAB, 16)])

    drain_idx(0, 0)
    issue(1)
    issue(2)
    for j in range(SLAB // ZCHUNK):
      pltpu.make_async_copy(buf0.at[pl.ds(0, ZCHUNK)],
                            acc.at[pl.ds(base + j * ZCHUNK, ZCHUNK)],
                            sem3).wait()
    issue(0)
    plsc.subcore_barrier()

    def step(j, v, last=False):
      b = v % NBUF
      par = (v // SK) % 2
      if not last:
        if v % SK == 0:
          fire_idx(j * (PERIOD // SK) + v // SK + 1, (par + 1) % 2)
        if v % SK == 2:
          drain_idx(j * (PERIOD // SK) + v // SK + 1, (par + 1) % 2)
        issue(v + 3)
      elif j * PERIOD + v + 3 < K:
        issue(v + 3)
      pltpu.make_async_copy(x_hbm.at[cidxs[par].at[v % SK]],
                            bufs[b], sems[b]).wait()
      pltpu.sync_copy(bufs[b], acc.at[ridxs[par].at[v % SK]], add=True)

    def body(j, carry):
      for v in range(PERIOD):
        step(j, v)
      return carry
    lax.fori_loop(0, K // PERIOD, body, 0)
    for v in range(PERIOD, PERIOD + SK):
      step(K // PERIOD - 1, v, last=True)

    plsc.subcore_barrier()
    pltpu.sync_copy(acc.at[pl.ds(base, SLAB)],
                    out_hbm.at[c, pl.ds(base, SLAB)])

    @pl.when(s == NS - 1)
    def _():
      pltpu.sync_copy(acc.at[pl.ds(NS * SLAB, 16)],
                      out_hbm.at[c, pl.ds(NS * SLAB, 16)])

  return sc_kernel(x, ei5)


def _tc_add(a, b):
  def add_kernel(a_ref, b_ref, o_ref):
    o_ref[...] = a_ref[...] + b_ref[...]

  return pl.pallas_call(
      add_kernel,
      out_shape=jax.ShapeDtypeStruct((N_NODES, D), jnp.float32),
  )(a, b)


@jax.jit
def kernel(x, edge_index):
  ei5 = edge_index.astype(jnp.int32).reshape(2, NW, STAGES, SK, CHUNK)
  partials = _sc_partial_sums(x, ei5)
  return _tc_add(partials[0], partials[1])
